# baseline (speedup 1.0000x reference)
import functools
import math

import jax
import jax.numpy as jnp
from jax.experimental import pallas as pl
from jax.experimental.pallas import tpu as pltpu


def _cross_pair(fs, ft, wc, wcat, a12, scale):
    hs = jnp.dot(fs, wc, preferred_element_type=jnp.float32)
    ht = jnp.dot(ft, wc, preferred_element_type=jnp.float32)
    sim = jax.lax.dot_general(hs, ht, (((1,), (1,)), ((), ())),
                              preferred_element_type=jnp.float32) * scale
    m1 = jnp.max(sim, axis=1, keepdims=True)
    p1 = jnp.exp(sim - m1)
    a_st = p1 / jnp.sum(p1, axis=1, keepdims=True)
    att_src = jnp.dot(a_st, ft, preferred_element_type=jnp.float32)
    m0 = jnp.max(sim, axis=0, keepdims=True)
    p0 = jnp.exp(sim - m0)
    a_ts = p0 / jnp.sum(p0, axis=0, keepdims=True)
    att_tgt = jax.lax.dot_general(a_ts, fs, (((0,), (0,)), ((), ())),
                                  preferred_element_type=jnp.float32)
    emb_s = jnp.concatenate([fs - att_src, fs], axis=1)
    emb_t = jnp.concatenate([ft - att_tgt, ft], axis=1)
    wh_s = jnp.dot(emb_s, wcat, preferred_element_type=jnp.float32)
    wh_t = jnp.dot(emb_t, wcat, preferred_element_type=jnp.float32)
    es = jnp.dot(wh_s, a12, preferred_element_type=jnp.float32)
    et = jnp.dot(wh_t, a12, preferred_element_type=jnp.float32)
    edt_s = jax.lax.dot_general(a12, wh_s, (((0,), (1,)), ((), ())),
                                preferred_element_type=jnp.float32)
    edt_t = jax.lax.dot_general(a12, wh_t, (((0,), (1,)), ((), ())),
                                preferred_element_type=jnp.float32)
    return sim, wh_s, wh_t, es, et, edt_s, edt_t


def _body(fs_ref, ft_ref, wc_ref, wcat_ref, a12_ref, ones_ref,
          gavg_ref, gmax_ref, nss_ref, nst_ref, adj_ref,
          out_ref, cross_ref, ns_ref,
          wh_vm, es_vm, edt_vm,
          *, scale, nheads, nhid, n, b_pairs):
    pid = pl.program_id(0)

    @pl.when(pid == 0)
    def _cross_stage():
        wc = wc_ref[...]
        wcat = wcat_ref[...]
        a12 = a12_ref[...]
        ones_col = ones_ref[...]
        for b in range(b_pairs):
            fs = fs_ref[b]
            ft = ft_ref[b]
            sim, wh_s, wh_t, es, et, edt_s, edt_t = _cross_pair(
                fs, ft, wc, wcat, a12, scale)
            wh_vm[2 * b * n:(2 * b + 1) * n] = (
                wh_s + ones_col).astype(jnp.bfloat16)
            wh_vm[(2 * b + 1) * n:(2 * b + 2) * n] = (
                wh_t + ones_col).astype(jnp.bfloat16)
            es_vm[2 * b * n:(2 * b + 1) * n] = es
            es_vm[(2 * b + 1) * n:(2 * b + 2) * n] = et
            edt_vm[:, 2 * b * n:(2 * b + 1) * n] = edt_s
            edt_vm[:, (2 * b + 1) * n:(2 * b + 2) * n] = edt_t
            cross_ref[b] = (gavg_ref[b] * jnp.mean(sim)
                            + gmax_ref[b] * jnp.max(sim))
            ns_ref[2 * b] = nss_ref[b]
            ns_ref[2 * b + 1] = nst_ref[b]

    side = pid % 2
    pair = pid // 2
    adj = adj_ref[...].astype(jnp.bfloat16)
    wh = wh_vm[...]
    es = es_vm[pl.ds(pid * n, n), :]
    orig = jnp.where(side == 0, fs_ref[pl.ds(pair, 1)],
                     ft_ref[pl.ds(pair, 1)])[0]
    edt = edt_vm[...]
    total = adj.shape[1]
    cm = jnp.dot(jnp.ones((8, total), jnp.bfloat16), wh,
                 preferred_element_type=jnp.float32)[0:1] * (1.0 / total)
    log2e = jnp.float32(1.4426950408889634)
    for h in range(nheads):
        ed = edt[h + nheads, :][None, :]
        es_h = es[:, h][:, None]
        u = es_h + jnp.max(ed)
        m = jnp.maximum(u, 0.2 * u)
        a_row = (es_h - m) * log2e
        b_row = (0.2 * es_h - m) * log2e
        ed1 = ed * log2e
        ed2 = ed * (0.2 * log2e)
        x = jnp.maximum(a_row + ed1, b_row + ed2)
        p = jnp.exp2(x).astype(jnp.bfloat16) * adj
        num_s = jnp.dot(p, wh[:, h * 128:(h + 1) * 128],
                        preferred_element_type=jnp.float32)
        num = num_s[:, :nhid]
        s = num_s[:, nhid:nhid + 1]
        pos = s > 0
        inv = 1.0 / jnp.where(pos, s, 1.0)
        upd = jnp.where(pos, num * inv, cm[:, h * 128:h * 128 + nhid])
        upd = jnp.where(upd > 0, upd, jnp.exp(jnp.minimum(upd, 0.0)) - 1.0)
        out_ref[:, h * nhid:(h + 1) * nhid] = (
            orig[:, h * nhid:(h + 1) * nhid] - upd)


def kernel(batch_feature_src, batch_feature_tgt, global_avg_weights,
           global_max_weights, ns_src, ns_tgt, adjacency_matrixs,
           W_cross, W_gat, a_gat):
    B, N, D = batch_feature_src.shape
    NHEADS, twoD, NHID = W_gat.shape
    HD = NHEADS * NHID
    TOTAL = 2 * B * N
    scale = 1.0 / math.sqrt(D)

    SLAB = 128
    W3 = jnp.transpose(W_gat, (1, 0, 2))
    W_cat = jnp.concatenate(
        [W3, jnp.zeros((twoD, NHEADS, SLAB - NHID), jnp.float32)],
        axis=2).reshape(twoD, NHEADS * SLAB)
    col = jnp.arange(NHEADS * SLAB)
    ones_row = ((col % SLAB) == NHID).astype(jnp.float32)[None, :]
    eye = jnp.eye(NHEADS, dtype=jnp.float32)
    T1 = eye[:, None, :] * a_gat[:, :NHID, None]
    T2 = eye[:, None, :] * a_gat[:, NHID:, None]
    zpad = jnp.zeros((NHEADS, SLAB - NHID, NHEADS), jnp.float32)
    A12 = jnp.concatenate(
        [jnp.concatenate([T1, zpad], axis=1),
         jnp.concatenate([T2, zpad], axis=1)],
        axis=2).reshape(NHEADS * SLAB, 2 * NHEADS)

    BLK = N
    out_node, cross_attention, ns = pl.pallas_call(
        functools.partial(_body, scale=scale, nheads=NHEADS, nhid=NHID,
                          n=N, b_pairs=B),
        grid=(TOTAL // BLK,),
        in_specs=[
            pl.BlockSpec((B, N, D), lambda i: (0, 0, 0)),
            pl.BlockSpec((B, N, D), lambda i: (0, 0, 0)),
            pl.BlockSpec((D, D), lambda i: (0, 0)),
            pl.BlockSpec((twoD, NHEADS * 128), lambda i: (0, 0)),
            pl.BlockSpec((NHEADS * 128, 2 * NHEADS), lambda i: (0, 0)),
            pl.BlockSpec((1, NHEADS * 128), lambda i: (0, 0)),
            pl.BlockSpec(memory_space=pltpu.SMEM),
            pl.BlockSpec(memory_space=pltpu.SMEM),
            pl.BlockSpec(memory_space=pltpu.SMEM),
            pl.BlockSpec(memory_space=pltpu.SMEM),
            pl.BlockSpec((BLK, TOTAL), lambda i: (i, 0)),
        ],
        out_specs=[
            pl.BlockSpec((BLK, D), lambda i: (i, 0)),
            pl.BlockSpec(memory_space=pltpu.SMEM),
            pl.BlockSpec(memory_space=pltpu.SMEM),
        ],
        out_shape=[
            jax.ShapeDtypeStruct((TOTAL, D), jnp.float32),
            jax.ShapeDtypeStruct((B,), jnp.float32),
            jax.ShapeDtypeStruct((2 * B,), jnp.int32),
        ],
        scratch_shapes=[
            pltpu.VMEM((TOTAL, NHEADS * 128), jnp.bfloat16),
            pltpu.VMEM((TOTAL, 2 * NHEADS), jnp.float32),
            pltpu.VMEM((2 * NHEADS, TOTAL), jnp.float32),
        ],
    )(batch_feature_src, batch_feature_tgt, W_cross, W_cat, A12, ones_row,
      global_avg_weights, global_max_weights, ns_src, ns_tgt,
      adjacency_matrixs)

    return cross_attention, out_node, ns


# speedup vs baseline: 1.1347x; 1.1347x over previous
import functools
import math

import jax
import jax.numpy as jnp
from jax.experimental import pallas as pl
from jax.experimental.pallas import tpu as pltpu


def _cross_pair(fs, ft, wc, wcat, a12, scale):
    hs = jnp.dot(fs, wc, preferred_element_type=jnp.float32)
    ht = jnp.dot(ft, wc, preferred_element_type=jnp.float32)
    sim = jax.lax.dot_general(hs, ht, (((1,), (1,)), ((), ())),
                              preferred_element_type=jnp.float32) * scale
    m1 = jnp.max(sim, axis=1, keepdims=True)
    p1 = jnp.exp(sim - m1)
    a_st = p1 / jnp.sum(p1, axis=1, keepdims=True)
    att_src = jnp.dot(a_st, ft, preferred_element_type=jnp.float32)
    m0 = jnp.max(sim, axis=0, keepdims=True)
    p0 = jnp.exp(sim - m0)
    a_ts = p0 / jnp.sum(p0, axis=0, keepdims=True)
    att_tgt = jax.lax.dot_general(a_ts, fs, (((0,), (0,)), ((), ())),
                                  preferred_element_type=jnp.float32)
    emb_s = jnp.concatenate([fs - att_src, fs], axis=1)
    emb_t = jnp.concatenate([ft - att_tgt, ft], axis=1)
    wh_s = jnp.dot(emb_s, wcat, preferred_element_type=jnp.float32)
    wh_t = jnp.dot(emb_t, wcat, preferred_element_type=jnp.float32)
    es = jnp.dot(wh_s, a12, preferred_element_type=jnp.float32)
    et = jnp.dot(wh_t, a12, preferred_element_type=jnp.float32)
    edt_s = jax.lax.dot_general(a12, wh_s, (((0,), (1,)), ((), ())),
                                preferred_element_type=jnp.float32)
    edt_t = jax.lax.dot_general(a12, wh_t, (((0,), (1,)), ((), ())),
                                preferred_element_type=jnp.float32)
    return sim, wh_s, wh_t, es, et, edt_s, edt_t


def _body(fs_ref, ft_ref, wc_ref, wcat_ref, a12_ref, ones_ref,
          gavg_ref, gmax_ref, nss_ref, nst_ref, adj_ref,
          out_ref, cross_ref, ns_ref,
          wh_vm, es_vm, edt_vm,
          *, scale, nheads, nhid, n, b_pairs):
    pid = pl.program_id(0)

    @pl.when(pid == 0)
    def _cross_stage():
        wc = wc_ref[...]
        wcat = wcat_ref[...]
        a12 = a12_ref[...]
        ones_col = ones_ref[...]
        for b in range(b_pairs):
            fs = fs_ref[b]
            ft = ft_ref[b]
            sim, wh_s, wh_t, es, et, edt_s, edt_t = _cross_pair(
                fs, ft, wc, wcat, a12, scale)
            wh_vm[2 * b * n:(2 * b + 1) * n] = (
                wh_s + ones_col).astype(jnp.bfloat16)
            wh_vm[(2 * b + 1) * n:(2 * b + 2) * n] = (
                wh_t + ones_col).astype(jnp.bfloat16)
            es_vm[2 * b * n:(2 * b + 1) * n] = es
            es_vm[(2 * b + 1) * n:(2 * b + 2) * n] = et
            edt_vm[:, 2 * b * n:(2 * b + 1) * n] = edt_s
            edt_vm[:, (2 * b + 1) * n:(2 * b + 2) * n] = edt_t
            cross_ref[b] = (gavg_ref[b] * jnp.mean(sim)
                            + gmax_ref[b] * jnp.max(sim))
            ns_ref[2 * b] = nss_ref[b]
            ns_ref[2 * b + 1] = nst_ref[b]

    side = pid % 2
    pair = pid // 2
    adj = adj_ref[...].astype(jnp.bfloat16)
    wh = wh_vm[...]
    es = es_vm[pl.ds(pid * n, n), :]
    orig = jnp.where(side == 0, fs_ref[pl.ds(pair, 1)],
                     ft_ref[pl.ds(pair, 1)])[0]
    edt = edt_vm[...]
    total = adj.shape[1]
    cm = jnp.dot(jnp.ones((8, total), jnp.bfloat16), wh,
                 preferred_element_type=jnp.float32)[0:1] * (1.0 / total)
    log2e = jnp.float32(1.4426950408889634)
    for h in range(nheads):
        ed = edt[h + nheads, :][None, :]
        es_h = es[:, h][:, None]
        u = es_h + jnp.max(ed)
        m = jnp.maximum(u, 0.2 * u)
        a_row = ((es_h - m) * log2e).astype(jnp.bfloat16)
        b_row = ((0.2 * es_h - m) * log2e).astype(jnp.bfloat16)
        ed1 = (ed * log2e).astype(jnp.bfloat16)
        ed2 = (ed * (0.2 * log2e)).astype(jnp.bfloat16)
        x = jnp.maximum(a_row + ed1, b_row + ed2)
        p = jnp.exp2(x) * adj
        num_s = jnp.dot(p, wh[:, h * 128:(h + 1) * 128],
                        preferred_element_type=jnp.float32)
        num = num_s[:, :nhid]
        s = num_s[:, nhid:nhid + 1]
        pos = s > 0
        inv = 1.0 / jnp.where(pos, s, 1.0)
        upd = jnp.where(pos, num * inv, cm[:, h * 128:h * 128 + nhid])
        upd = jnp.where(upd > 0, upd, jnp.exp(jnp.minimum(upd, 0.0)) - 1.0)
        out_ref[:, h * nhid:(h + 1) * nhid] = (
            orig[:, h * nhid:(h + 1) * nhid] - upd)


def kernel(batch_feature_src, batch_feature_tgt, global_avg_weights,
           global_max_weights, ns_src, ns_tgt, adjacency_matrixs,
           W_cross, W_gat, a_gat):
    B, N, D = batch_feature_src.shape
    NHEADS, twoD, NHID = W_gat.shape
    HD = NHEADS * NHID
    TOTAL = 2 * B * N
    scale = 1.0 / math.sqrt(D)

    SLAB = 128
    W3 = jnp.transpose(W_gat, (1, 0, 2))
    W_cat = jnp.concatenate(
        [W3, jnp.zeros((twoD, NHEADS, SLAB - NHID), jnp.float32)],
        axis=2).reshape(twoD, NHEADS * SLAB)
    col = jnp.arange(NHEADS * SLAB)
    ones_row = ((col % SLAB) == NHID).astype(jnp.float32)[None, :]
    eye = jnp.eye(NHEADS, dtype=jnp.float32)
    T1 = eye[:, None, :] * a_gat[:, :NHID, None]
    T2 = eye[:, None, :] * a_gat[:, NHID:, None]
    zpad = jnp.zeros((NHEADS, SLAB - NHID, NHEADS), jnp.float32)
    A12 = jnp.concatenate(
        [jnp.concatenate([T1, zpad], axis=1),
         jnp.concatenate([T2, zpad], axis=1)],
        axis=2).reshape(NHEADS * SLAB, 2 * NHEADS)

    BLK = N
    out_node, cross_attention, ns = pl.pallas_call(
        functools.partial(_body, scale=scale, nheads=NHEADS, nhid=NHID,
                          n=N, b_pairs=B),
        grid=(TOTAL // BLK,),
        in_specs=[
            pl.BlockSpec((B, N, D), lambda i: (0, 0, 0)),
            pl.BlockSpec((B, N, D), lambda i: (0, 0, 0)),
            pl.BlockSpec((D, D), lambda i: (0, 0)),
            pl.BlockSpec((twoD, NHEADS * 128), lambda i: (0, 0)),
            pl.BlockSpec((NHEADS * 128, 2 * NHEADS), lambda i: (0, 0)),
            pl.BlockSpec((1, NHEADS * 128), lambda i: (0, 0)),
            pl.BlockSpec(memory_space=pltpu.SMEM),
            pl.BlockSpec(memory_space=pltpu.SMEM),
            pl.BlockSpec(memory_space=pltpu.SMEM),
            pl.BlockSpec(memory_space=pltpu.SMEM),
            pl.BlockSpec((BLK, TOTAL), lambda i: (i, 0)),
        ],
        out_specs=[
            pl.BlockSpec((BLK, D), lambda i: (i, 0)),
            pl.BlockSpec(memory_space=pltpu.SMEM),
            pl.BlockSpec(memory_space=pltpu.SMEM),
        ],
        out_shape=[
            jax.ShapeDtypeStruct((TOTAL, D), jnp.float32),
            jax.ShapeDtypeStruct((B,), jnp.float32),
            jax.ShapeDtypeStruct((2 * B,), jnp.int32),
        ],
        scratch_shapes=[
            pltpu.VMEM((TOTAL, NHEADS * 128), jnp.bfloat16),
            pltpu.VMEM((TOTAL, 2 * NHEADS), jnp.float32),
            pltpu.VMEM((2 * NHEADS, TOTAL), jnp.float32),
        ],
    )(batch_feature_src, batch_feature_tgt, W_cross, W_cat, A12, ones_row,
      global_avg_weights, global_max_weights, ns_src, ns_tgt,
      adjacency_matrixs)

    return cross_attention, out_node, ns


# speedup vs baseline: 1.1400x; 1.0046x over previous
import functools
import math

import jax
import jax.numpy as jnp
from jax.experimental import pallas as pl
from jax.experimental.pallas import tpu as pltpu


def _cross_pair(fs, ft, wc, wcat, a12, scale):
    hs = jnp.dot(fs, wc, preferred_element_type=jnp.float32)
    ht = jnp.dot(ft, wc, preferred_element_type=jnp.float32)
    sim = jax.lax.dot_general(hs, ht, (((1,), (1,)), ((), ())),
                              preferred_element_type=jnp.float32) * scale
    m1 = jnp.max(sim, axis=1, keepdims=True)
    p1 = jnp.exp(sim - m1)
    a_st = p1 / jnp.sum(p1, axis=1, keepdims=True)
    fs_bf = fs.astype(jnp.bfloat16)
    ft_bf = ft.astype(jnp.bfloat16)
    att_src = jnp.dot(a_st.astype(jnp.bfloat16), ft_bf,
                      preferred_element_type=jnp.float32)
    m0 = jnp.max(sim, axis=0, keepdims=True)
    p0 = jnp.exp(sim - m0)
    a_ts = p0 / jnp.sum(p0, axis=0, keepdims=True)
    att_tgt = jax.lax.dot_general(a_ts.astype(jnp.bfloat16), fs_bf,
                                  (((0,), (0,)), ((), ())),
                                  preferred_element_type=jnp.float32)
    emb_s = jnp.concatenate([fs - att_src, fs], axis=1)
    emb_t = jnp.concatenate([ft - att_tgt, ft], axis=1)
    wh_s = jnp.dot(emb_s.astype(jnp.bfloat16), wcat,
                   preferred_element_type=jnp.float32)
    wh_t = jnp.dot(emb_t.astype(jnp.bfloat16), wcat,
                   preferred_element_type=jnp.float32)
    es = jnp.dot(wh_s, a12, preferred_element_type=jnp.float32)
    et = jnp.dot(wh_t, a12, preferred_element_type=jnp.float32)
    edt_s = jax.lax.dot_general(a12, wh_s, (((0,), (1,)), ((), ())),
                                preferred_element_type=jnp.float32)
    edt_t = jax.lax.dot_general(a12, wh_t, (((0,), (1,)), ((), ())),
                                preferred_element_type=jnp.float32)
    return sim, wh_s, wh_t, es, et, edt_s, edt_t


def _body(fs_ref, ft_ref, wc_ref, wcat_ref, a12_ref, ones_ref,
          gavg_ref, gmax_ref, nss_ref, nst_ref, adj_ref,
          out_ref, cross_ref, ns_ref,
          wh_vm, es_vm, edt_vm,
          *, scale, nheads, nhid, n, b_pairs):
    pid = pl.program_id(0)

    @pl.when(pid == 0)
    def _cross_stage():
        wc = wc_ref[...]
        wcat = wcat_ref[...]
        a12 = a12_ref[...]
        ones_col = ones_ref[...]
        for b in range(b_pairs):
            fs = fs_ref[b]
            ft = ft_ref[b]
            sim, wh_s, wh_t, es, et, edt_s, edt_t = _cross_pair(
                fs, ft, wc, wcat, a12, scale)
            wh_vm[2 * b * n:(2 * b + 1) * n] = (
                wh_s + ones_col).astype(jnp.bfloat16)
            wh_vm[(2 * b + 1) * n:(2 * b + 2) * n] = (
                wh_t + ones_col).astype(jnp.bfloat16)
            es_vm[2 * b * n:(2 * b + 1) * n] = es
            es_vm[(2 * b + 1) * n:(2 * b + 2) * n] = et
            edt_vm[:, 2 * b * n:(2 * b + 1) * n] = edt_s
            edt_vm[:, (2 * b + 1) * n:(2 * b + 2) * n] = edt_t
            cross_ref[b] = (gavg_ref[b] * jnp.mean(sim)
                            + gmax_ref[b] * jnp.max(sim))
            ns_ref[2 * b] = nss_ref[b]
            ns_ref[2 * b + 1] = nst_ref[b]

    side = pid % 2
    pair = pid // 2
    adj = adj_ref[...].astype(jnp.bfloat16)
    wh = wh_vm[...]
    es = es_vm[pl.ds(pid * n, n), :]
    orig = jnp.where(side == 0, fs_ref[pl.ds(pair, 1)],
                     ft_ref[pl.ds(pair, 1)])[0]
    edt = edt_vm[...]
    total = adj.shape[1]
    cm = jnp.dot(jnp.ones((8, total), jnp.bfloat16), wh,
                 preferred_element_type=jnp.float32)[0:1] * (1.0 / total)
    log2e = jnp.float32(1.4426950408889634)
    for h in range(nheads):
        ed = edt[h + nheads, :][None, :]
        es_h = es[:, h][:, None]
        u = es_h + jnp.max(ed)
        m = jnp.maximum(u, 0.2 * u)
        a_row = ((es_h - m) * log2e).astype(jnp.bfloat16)
        b_row = ((0.2 * es_h - m) * log2e).astype(jnp.bfloat16)
        ed1 = (ed * log2e).astype(jnp.bfloat16)
        ed2 = (ed * (0.2 * log2e)).astype(jnp.bfloat16)
        x = jnp.maximum(a_row + ed1, b_row + ed2)
        p = jnp.exp2(x) * adj
        num_s = jnp.dot(p, wh[:, h * 128:(h + 1) * 128],
                        preferred_element_type=jnp.float32)
        num = num_s[:, :nhid]
        s = num_s[:, nhid:nhid + 1]
        pos = s > 0
        inv = 1.0 / jnp.where(pos, s, 1.0)
        upd = jnp.where(pos, num * inv, cm[:, h * 128:h * 128 + nhid])
        upd = jnp.where(upd > 0, upd, jnp.exp(jnp.minimum(upd, 0.0)) - 1.0)
        out_ref[:, h * nhid:(h + 1) * nhid] = (
            orig[:, h * nhid:(h + 1) * nhid] - upd)


def kernel(batch_feature_src, batch_feature_tgt, global_avg_weights,
           global_max_weights, ns_src, ns_tgt, adjacency_matrixs,
           W_cross, W_gat, a_gat):
    B, N, D = batch_feature_src.shape
    NHEADS, twoD, NHID = W_gat.shape
    HD = NHEADS * NHID
    TOTAL = 2 * B * N
    scale = 1.0 / math.sqrt(D)

    SLAB = 128
    W3 = jnp.transpose(W_gat, (1, 0, 2))
    W_cat = jnp.concatenate(
        [W3, jnp.zeros((twoD, NHEADS, SLAB - NHID), jnp.float32)],
        axis=2).reshape(twoD, NHEADS * SLAB).astype(jnp.bfloat16)
    col = jnp.arange(NHEADS * SLAB)
    ones_row = ((col % SLAB) == NHID).astype(jnp.float32)[None, :]
    eye = jnp.eye(NHEADS, dtype=jnp.float32)
    T1 = eye[:, None, :] * a_gat[:, :NHID, None]
    T2 = eye[:, None, :] * a_gat[:, NHID:, None]
    zpad = jnp.zeros((NHEADS, SLAB - NHID, NHEADS), jnp.float32)
    A12 = jnp.concatenate(
        [jnp.concatenate([T1, zpad], axis=1),
         jnp.concatenate([T2, zpad], axis=1)],
        axis=2).reshape(NHEADS * SLAB, 2 * NHEADS)

    BLK = N
    out_node, cross_attention, ns = pl.pallas_call(
        functools.partial(_body, scale=scale, nheads=NHEADS, nhid=NHID,
                          n=N, b_pairs=B),
        grid=(TOTAL // BLK,),
        in_specs=[
            pl.BlockSpec((B, N, D), lambda i: (0, 0, 0)),
            pl.BlockSpec((B, N, D), lambda i: (0, 0, 0)),
            pl.BlockSpec((D, D), lambda i: (0, 0)),
            pl.BlockSpec((twoD, NHEADS * 128), lambda i: (0, 0)),
            pl.BlockSpec((NHEADS * 128, 2 * NHEADS), lambda i: (0, 0)),
            pl.BlockSpec((1, NHEADS * 128), lambda i: (0, 0)),
            pl.BlockSpec(memory_space=pltpu.SMEM),
            pl.BlockSpec(memory_space=pltpu.SMEM),
            pl.BlockSpec(memory_space=pltpu.SMEM),
            pl.BlockSpec(memory_space=pltpu.SMEM),
            pl.BlockSpec((BLK, TOTAL), lambda i: (i, 0)),
        ],
        out_specs=[
            pl.BlockSpec((BLK, D), lambda i: (i, 0)),
            pl.BlockSpec(memory_space=pltpu.SMEM),
            pl.BlockSpec(memory_space=pltpu.SMEM),
        ],
        out_shape=[
            jax.ShapeDtypeStruct((TOTAL, D), jnp.float32),
            jax.ShapeDtypeStruct((B,), jnp.float32),
            jax.ShapeDtypeStruct((2 * B,), jnp.int32),
        ],
        scratch_shapes=[
            pltpu.VMEM((TOTAL, NHEADS * 128), jnp.bfloat16),
            pltpu.VMEM((TOTAL, 2 * NHEADS), jnp.float32),
            pltpu.VMEM((2 * NHEADS, TOTAL), jnp.float32),
        ],
    )(batch_feature_src, batch_feature_tgt, W_cross, W_cat, A12, ones_row,
      global_avg_weights, global_max_weights, ns_src, ns_tgt,
      adjacency_matrixs)

    return cross_attention, out_node, ns


# speedup vs baseline: 1.1704x; 1.0267x over previous
import functools
import math

import jax
import jax.numpy as jnp
from jax.experimental import pallas as pl
from jax.experimental.pallas import tpu as pltpu


def _cross_pair(fs, ft, wc, wcat, a12, scale):
    hs = jnp.dot(fs, wc, preferred_element_type=jnp.float32)
    ht = jnp.dot(ft, wc, preferred_element_type=jnp.float32)
    sim = jax.lax.dot_general(hs, ht, (((1,), (1,)), ((), ())),
                              preferred_element_type=jnp.float32) * scale
    m1 = jnp.max(sim, axis=1, keepdims=True)
    p1 = jnp.exp(sim - m1)
    a_st = p1 / jnp.sum(p1, axis=1, keepdims=True)
    fs_bf = fs.astype(jnp.bfloat16)
    ft_bf = ft.astype(jnp.bfloat16)
    att_src = jnp.dot(a_st.astype(jnp.bfloat16), ft_bf,
                      preferred_element_type=jnp.float32)
    m0 = jnp.max(sim, axis=0, keepdims=True)
    p0 = jnp.exp(sim - m0)
    a_ts = p0 / jnp.sum(p0, axis=0, keepdims=True)
    att_tgt = jax.lax.dot_general(a_ts.astype(jnp.bfloat16), fs_bf,
                                  (((0,), (0,)), ((), ())),
                                  preferred_element_type=jnp.float32)
    emb_s = jnp.concatenate([fs - att_src, fs], axis=1)
    emb_t = jnp.concatenate([ft - att_tgt, ft], axis=1)
    wh_s = jnp.dot(emb_s.astype(jnp.bfloat16), wcat,
                   preferred_element_type=jnp.float32)
    wh_t = jnp.dot(emb_t.astype(jnp.bfloat16), wcat,
                   preferred_element_type=jnp.float32)
    es = jnp.dot(wh_s, a12, preferred_element_type=jnp.float32)
    et = jnp.dot(wh_t, a12, preferred_element_type=jnp.float32)
    edt_s = jax.lax.dot_general(a12, wh_s, (((0,), (1,)), ((), ())),
                                preferred_element_type=jnp.float32)
    edt_t = jax.lax.dot_general(a12, wh_t, (((0,), (1,)), ((), ())),
                                preferred_element_type=jnp.float32)
    return sim, wh_s, wh_t, es, et, edt_s, edt_t


def _body(fs_ref, ft_ref, wc_ref, wcat_ref, a12_ref, ones_ref,
          gavg_ref, gmax_ref, nss_ref, nst_ref, adj_ref,
          out_ref, cross_ref, ns_ref,
          wh_vm, es_vm, edt_vm, cm_vm,
          *, scale, nheads, nhid, n, b_pairs):
    pid = pl.program_id(0)

    @pl.when(pid == 0)
    def _cross_stage():
        wc = wc_ref[...]
        wcat = wcat_ref[...]
        a12 = a12_ref[...]
        ones_col = ones_ref[...]
        for b in range(b_pairs):
            fs = fs_ref[b]
            ft = ft_ref[b]
            sim, wh_s, wh_t, es, et, edt_s, edt_t = _cross_pair(
                fs, ft, wc, wcat, a12, scale)
            wh_vm[2 * b * n:(2 * b + 1) * n] = (
                wh_s + ones_col).astype(jnp.bfloat16)
            wh_vm[(2 * b + 1) * n:(2 * b + 2) * n] = (
                wh_t + ones_col).astype(jnp.bfloat16)
            es_vm[2 * b * n:(2 * b + 1) * n] = es
            es_vm[(2 * b + 1) * n:(2 * b + 2) * n] = et
            edt_vm[:, 2 * b * n:(2 * b + 1) * n] = edt_s
            edt_vm[:, (2 * b + 1) * n:(2 * b + 2) * n] = edt_t
            cross_ref[b] = (gavg_ref[b] * jnp.mean(sim)
                            + gmax_ref[b] * jnp.max(sim))
            ns_ref[2 * b] = nss_ref[b]
            ns_ref[2 * b + 1] = nst_ref[b]
        total = 2 * b_pairs * n
        cm_vm[...] = jnp.dot(jnp.ones((8, total), jnp.bfloat16), wh_vm[...],
                             preferred_element_type=jnp.float32) * (1.0 / total)

    side = pid % 2
    pair = pid // 2
    adj = adj_ref[...].astype(jnp.bfloat16)
    wh = wh_vm[...]
    es = es_vm[pl.ds(pid * n, n), :]
    orig = jnp.where(side == 0, fs_ref[pl.ds(pair, 1)],
                     ft_ref[pl.ds(pair, 1)])[0]
    edt = edt_vm[...]
    cm = cm_vm[0:1]
    log2e = jnp.float32(1.4426950408889634)
    for h in range(nheads):
        ed = edt[h + nheads, :][None, :]
        es_h = es[:, h][:, None]
        u = es_h + jnp.max(ed)
        m = jnp.maximum(u, 0.2 * u)
        a_row = ((es_h - m) * log2e).astype(jnp.bfloat16)
        b_row = ((0.2 * es_h - m) * log2e).astype(jnp.bfloat16)
        ed1 = (ed * log2e).astype(jnp.bfloat16)
        ed2 = (ed * (0.2 * log2e)).astype(jnp.bfloat16)
        x = jnp.maximum(a_row + ed1, b_row + ed2)
        p = jnp.exp2(x) * adj
        num_s = jnp.dot(p, wh[:, h * 128:(h + 1) * 128],
                        preferred_element_type=jnp.float32)
        num = num_s[:, :nhid]
        s = num_s[:, nhid:nhid + 1]
        pos = s > 0
        inv = 1.0 / jnp.where(pos, s, 1.0)
        upd = jnp.where(pos, num * inv, cm[:, h * 128:h * 128 + nhid])
        upd = jnp.where(upd > 0, upd, jnp.exp(jnp.minimum(upd, 0.0)) - 1.0)
        out_ref[:, h * nhid:(h + 1) * nhid] = (
            orig[:, h * nhid:(h + 1) * nhid] - upd)


def kernel(batch_feature_src, batch_feature_tgt, global_avg_weights,
           global_max_weights, ns_src, ns_tgt, adjacency_matrixs,
           W_cross, W_gat, a_gat):
    B, N, D = batch_feature_src.shape
    NHEADS, twoD, NHID = W_gat.shape
    HD = NHEADS * NHID
    TOTAL = 2 * B * N
    scale = 1.0 / math.sqrt(D)

    SLAB = 128
    W3 = jnp.transpose(W_gat, (1, 0, 2))
    W_cat = jnp.concatenate(
        [W3, jnp.zeros((twoD, NHEADS, SLAB - NHID), jnp.float32)],
        axis=2).reshape(twoD, NHEADS * SLAB).astype(jnp.bfloat16)
    col = jnp.arange(NHEADS * SLAB)
    ones_row = ((col % SLAB) == NHID).astype(jnp.float32)[None, :]
    eye = jnp.eye(NHEADS, dtype=jnp.float32)
    T1 = eye[:, None, :] * a_gat[:, :NHID, None]
    T2 = eye[:, None, :] * a_gat[:, NHID:, None]
    zpad = jnp.zeros((NHEADS, SLAB - NHID, NHEADS), jnp.float32)
    A12 = jnp.concatenate(
        [jnp.concatenate([T1, zpad], axis=1),
         jnp.concatenate([T2, zpad], axis=1)],
        axis=2).reshape(NHEADS * SLAB, 2 * NHEADS)

    BLK = N
    out_node, cross_attention, ns = pl.pallas_call(
        functools.partial(_body, scale=scale, nheads=NHEADS, nhid=NHID,
                          n=N, b_pairs=B),
        grid=(TOTAL // BLK,),
        in_specs=[
            pl.BlockSpec((B, N, D), lambda i: (0, 0, 0)),
            pl.BlockSpec((B, N, D), lambda i: (0, 0, 0)),
            pl.BlockSpec((D, D), lambda i: (0, 0)),
            pl.BlockSpec((twoD, NHEADS * 128), lambda i: (0, 0)),
            pl.BlockSpec((NHEADS * 128, 2 * NHEADS), lambda i: (0, 0)),
            pl.BlockSpec((1, NHEADS * 128), lambda i: (0, 0)),
            pl.BlockSpec(memory_space=pltpu.SMEM),
            pl.BlockSpec(memory_space=pltpu.SMEM),
            pl.BlockSpec(memory_space=pltpu.SMEM),
            pl.BlockSpec(memory_space=pltpu.SMEM),
            pl.BlockSpec((BLK, TOTAL), lambda i: (i, 0)),
        ],
        out_specs=[
            pl.BlockSpec((BLK, D), lambda i: (i, 0)),
            pl.BlockSpec(memory_space=pltpu.SMEM),
            pl.BlockSpec(memory_space=pltpu.SMEM),
        ],
        out_shape=[
            jax.ShapeDtypeStruct((TOTAL, D), jnp.float32),
            jax.ShapeDtypeStruct((B,), jnp.float32),
            jax.ShapeDtypeStruct((2 * B,), jnp.int32),
        ],
        scratch_shapes=[
            pltpu.VMEM((TOTAL, NHEADS * 128), jnp.bfloat16),
            pltpu.VMEM((TOTAL, 2 * NHEADS), jnp.float32),
            pltpu.VMEM((2 * NHEADS, TOTAL), jnp.float32),
            pltpu.VMEM((8, NHEADS * 128), jnp.float32),
        ],
    )(batch_feature_src, batch_feature_tgt, W_cross, W_cat, A12, ones_row,
      global_avg_weights, global_max_weights, ns_src, ns_tgt,
      adjacency_matrixs)

    return cross_attention, out_node, ns
